# Initial kernel scaffold; baseline (speedup 1.0000x reference)
import jax
import jax.numpy as jnp
from jax import lax
from jax.experimental import pallas as pl
from jax.experimental.pallas import tpu as pltpu
from jax.experimental.pallas import tpu_sc as plsc

N = 10000
E = 160000
H = 2
D = 256
DP = 128
NC = 2
NS = 16
L = 16
EPT = E // NS
KB = 80
NBLK = EPT // KB
RPT = N // NS

_MESH = plsc.VectorSubcoreMesh(core_axis_name="c", subcore_axis_name="s",
                               num_cores=NC, num_subcores=NS)


def _zero_rows(ref, nrows, ncols):
    z = jnp.zeros((L,), jnp.float32)

    def body(i, _):
        for c in range(ncols // L):
            ref[i, pl.ds(c * L, L)] = z
        return 0

    lax.fori_loop(0, nrows, body, 0)


def _phase_a(src2, dst2, elr, attr2, aet):
    def body(src_hbm, dst_hbm, elr_hbm, attr_hbm, aet_hbm,
             den_hbm, ex_hbm,
             den_sh, zbuf, sidx, didx, dgidx, srows, drows, attrb,
             ee0b, ee1b, ex0b, ex1b, exw, aeb, sem):
        g = lax.axis_index("c")
        sid = lax.axis_index("s")
        iota = lax.iota(jnp.int32, L)
        c0 = jnp.zeros((L,), jnp.int32)
        c1 = c0 + 1
        c2 = c0 + 2
        c3 = c0 + 3

        _zero_rows(zbuf, RPT, L)
        _zero_rows(exw, KB, L)
        pltpu.sync_copy(zbuf, den_sh.at[pl.ds(sid * RPT, RPT), :])
        pltpu.sync_copy(aet_hbm, aeb)
        plsc.subcore_barrier()

        ae0 = aeb[0, :]
        ae1 = aeb[1, :]

        def block(b, _):
            base = g * E + sid * EPT + b * KB
            pltpu.sync_copy(src_hbm.at[pl.ds(base, KB)], sidx)
            pltpu.sync_copy(dst_hbm.at[pl.ds(base, KB)], didx)
            pltpu.sync_copy(attr_hbm.at[pl.ds(base, KB), :], attrb)
            for q in range(KB // L):
                dv = didx[pl.ds(q * L, L)]
                dgidx[pl.ds(q * L, L)] = dv + g * N
            pltpu.async_copy(elr_hbm.at[sidx], srows, sem).wait()
            pltpu.async_copy(elr_hbm.at[dgidx], drows, sem).wait()

            def edge(j, _):
                av = attrb[j, :]
                ee0b[j] = jnp.sum(av * ae0)
                ee1b[j] = jnp.sum(av * ae1)
                return 0

            lax.fori_loop(0, KB, edge, 0)

            for q in range(KB // L):
                jq = iota + q * L
                el0 = plsc.load_gather(srows, [jq, c0])
                el1 = plsc.load_gather(srows, [jq, c1])
                er0 = plsc.load_gather(drows, [jq, c2])
                er1 = plsc.load_gather(drows, [jq, c3])
                s0 = el0 + er0 + ee0b[pl.ds(q * L, L)]
                s1 = el1 + er1 + ee1b[pl.ds(q * L, L)]
                s0 = jnp.maximum(s0, 0.2 * s0)
                s1 = jnp.maximum(s1, 0.2 * s1)
                x0 = jnp.exp(s0)
                x1 = jnp.exp(s1)
                ex0b[pl.ds(q * L, L)] = x0
                ex1b[pl.ds(q * L, L)] = x1
                plsc.store_scatter(exw, [jq, c0], x0)
                plsc.store_scatter(exw, [jq, c1], x1)
            pltpu.sync_copy(exw, den_sh.at[didx], add=True)
            pltpu.sync_copy(ex0b, ex_hbm.at[0, pl.ds(base, KB)])
            pltpu.sync_copy(ex1b, ex_hbm.at[1, pl.ds(base, KB)])
            return 0

        lax.fori_loop(0, NBLK, block, 0)
        plsc.subcore_barrier()
        pltpu.sync_copy(den_sh.at[pl.ds(sid * RPT, RPT), :],
                        den_hbm.at[pl.ds(g * N + sid * RPT, RPT), :])

    f = pl.kernel(
        body,
        out_type=[
            jax.ShapeDtypeStruct((2 * N, L), jnp.float32),
            jax.ShapeDtypeStruct((2, 2 * E), jnp.float32),
        ],
        mesh=_MESH,
        scratch_types=[
            pltpu.VMEM_SHARED((N, L), jnp.float32),
            pltpu.VMEM((RPT, L), jnp.float32),
            pltpu.VMEM((KB,), jnp.int32),
            pltpu.VMEM((KB,), jnp.int32),
            pltpu.VMEM((KB,), jnp.int32),
            pltpu.VMEM((KB, 8), jnp.float32),
            pltpu.VMEM((KB, 8), jnp.float32),
            pltpu.VMEM((KB, L), jnp.float32),
            pltpu.VMEM((KB,), jnp.float32),
            pltpu.VMEM((KB,), jnp.float32),
            pltpu.VMEM((KB,), jnp.float32),
            pltpu.VMEM((KB,), jnp.float32),
            pltpu.VMEM((KB, L), jnp.float32),
            pltpu.VMEM((2, L), jnp.float32),
            pltpu.SemaphoreType.DMA,
        ],
    )
    return f(src2, dst2, elr, attr2, aet)


def _phase_b(src2, dst2, featv, ex, den):
    def body(src_hbm, dst_hbm, feat_hbm, ex_hbm, den_hbm,
             h_hbm,
             acc_sh, zbuf, sidx, didx, dgidx, fi0, fi1,
             f0b, f1b, denb, ex0b, ex1b, al0b, al1b, msg, sem):
        p = lax.axis_index("c")
        sid = lax.axis_index("s")
        iota = lax.iota(jnp.int32, L)
        c0 = jnp.zeros((L,), jnp.int32)
        c1 = c0 + 1

        _zero_rows(zbuf, RPT // 5, DP)

        for g in range(2):
            for z in range(5):
                pltpu.sync_copy(
                    zbuf,
                    acc_sh.at[pl.ds(sid * RPT + z * (RPT // 5), RPT // 5), :])
            plsc.subcore_barrier()

            def block(b, _):
                base = g * E + sid * EPT + b * KB
                pltpu.sync_copy(src_hbm.at[pl.ds(base, KB)], sidx)
                pltpu.sync_copy(dst_hbm.at[pl.ds(base, KB)], didx)
                pltpu.sync_copy(ex_hbm.at[0, pl.ds(base, KB)], ex0b)
                pltpu.sync_copy(ex_hbm.at[1, pl.ds(base, KB)], ex1b)
                for q in range(KB // L):
                    sv = sidx[pl.ds(q * L, L)]
                    f0 = sv * 4 + p
                    fi0[pl.ds(q * L, L)] = f0
                    fi1[pl.ds(q * L, L)] = f0 + 2
                    dv = didx[pl.ds(q * L, L)]
                    dgidx[pl.ds(q * L, L)] = dv + g * N
                pltpu.async_copy(feat_hbm.at[fi0], f0b, sem).wait()
                pltpu.async_copy(feat_hbm.at[fi1], f1b, sem).wait()
                pltpu.async_copy(den_hbm.at[dgidx], denb, sem).wait()
                for q in range(KB // L):
                    jq = iota + q * L
                    d0 = plsc.load_gather(denb, [jq, c0])
                    d1 = plsc.load_gather(denb, [jq, c1])
                    a0 = ex0b[pl.ds(q * L, L)] * (0.5 / (d0 + 1e-9))
                    a1 = ex1b[pl.ds(q * L, L)] * (0.5 / (d1 + 1e-9))
                    al0b[pl.ds(q * L, L)] = a0
                    al1b[pl.ds(q * L, L)] = a1

                def edge(j, _):
                    a0 = al0b[j]
                    a1 = al1b[j]
                    for c in range(DP // L):
                        sl = pl.ds(c * L, L)
                        msg[j, sl] = a0 * f0b[j, sl] + a1 * f1b[j, sl]
                    return 0

                lax.fori_loop(0, KB, edge, 0)
                pltpu.sync_copy(msg, acc_sh.at[didx], add=True)
                return 0

            lax.fori_loop(0, NBLK, block, 0)
            plsc.subcore_barrier()
            pltpu.sync_copy(
                acc_sh.at[pl.ds(sid * RPT, RPT), :],
                h_hbm.at[p, pl.ds(g * N + sid * RPT, RPT), :])
            plsc.subcore_barrier()

    f = pl.kernel(
        body,
        out_type=[
            jax.ShapeDtypeStruct((NC, 2 * N, DP), jnp.float32),
        ],
        mesh=_MESH,
        scratch_types=[
            pltpu.VMEM_SHARED((N, DP), jnp.float32),
            pltpu.VMEM((RPT // 5, DP), jnp.float32),
            pltpu.VMEM((KB,), jnp.int32),
            pltpu.VMEM((KB,), jnp.int32),
            pltpu.VMEM((KB,), jnp.int32),
            pltpu.VMEM((KB,), jnp.int32),
            pltpu.VMEM((KB,), jnp.int32),
            pltpu.VMEM((KB, DP), jnp.float32),
            pltpu.VMEM((KB, DP), jnp.float32),
            pltpu.VMEM((KB, L), jnp.float32),
            pltpu.VMEM((KB,), jnp.float32),
            pltpu.VMEM((KB,), jnp.float32),
            pltpu.VMEM((KB,), jnp.float32),
            pltpu.VMEM((KB,), jnp.float32),
            pltpu.VMEM((KB, DP), jnp.float32),
            pltpu.SemaphoreType.DMA,
        ],
    )
    return f(src2, dst2, featv, ex, den)


def _mm_flat(x2, wb, b2, bm=2000):
    n = x2.shape[0]

    def body(x_ref, w_ref, b2_ref, feat_ref, elr_ref):
        x = x_ref[...]
        feat_ref[...] = jnp.dot(x, w_ref[...], preferred_element_type=jnp.float32)
        elr_ref[...] = jnp.dot(x, b2_ref[...], preferred_element_type=jnp.float32)

    return pl.pallas_call(
        body,
        grid=(n // bm,),
        in_specs=[
            pl.BlockSpec((bm, D), lambda i: (i, 0)),
            pl.BlockSpec((D, H * D), lambda i: (0, 0)),
            pl.BlockSpec((D, 8), lambda i: (0, 0)),
        ],
        out_specs=[
            pl.BlockSpec((bm, H * D), lambda i: (i, 0)),
            pl.BlockSpec((bm, 8), lambda i: (i, 0)),
        ],
        out_shape=[
            jax.ShapeDtypeStruct((n, H * D), jnp.float32),
            jax.ShapeDtypeStruct((n, 8), jnp.float32),
        ],
    )(x2, wb, b2)


def _mm_planes(hp, wb, b2, bm=2000):
    n = hp.shape[1]

    def body(x_ref, w_ref, b2_ref, feat_ref, elr_ref):
        x = jnp.concatenate([x_ref[0], x_ref[1]], axis=-1)
        feat_ref[...] = jnp.dot(x, w_ref[...], preferred_element_type=jnp.float32)
        elr_ref[...] = jnp.dot(x, b2_ref[...], preferred_element_type=jnp.float32)

    return pl.pallas_call(
        body,
        grid=(n // bm,),
        in_specs=[
            pl.BlockSpec((NC, bm, DP), lambda i: (0, i, 0)),
            pl.BlockSpec((D, H * D), lambda i: (0, 0)),
            pl.BlockSpec((D, 8), lambda i: (0, 0)),
        ],
        out_specs=[
            pl.BlockSpec((bm, H * D), lambda i: (i, 0)),
            pl.BlockSpec((bm, 8), lambda i: (i, 0)),
        ],
        out_shape=[
            jax.ShapeDtypeStruct((n, H * D), jnp.float32),
            jax.ShapeDtypeStruct((n, 8), jnp.float32),
        ],
    )(hp, wb, b2)


def _final(hp, wc1, bc1, wc2p, bc2p, bm=2000):
    nblk = 2 * N // bm
    half = nblk // 2

    def body(x_ref, w1_ref, b1_ref, w2_ref, b2_ref, out_ref, acc):
        i = pl.program_id(0)

        @pl.when(i == 0)
        def _():
            acc[...] = jnp.zeros_like(acc)

        x = jnp.concatenate([x_ref[0], x_ref[1]], axis=-1)
        s = jnp.sum(x, axis=0, keepdims=True)
        is_img = (i < half).astype(jnp.float32)
        acc[0:1, :] += is_img * s
        acc[1:2, :] += (1.0 - is_img) * s

        @pl.when(i == nblk - 1)
        def _():
            gcat = jnp.concatenate([acc[0:1, :], acc[1:2, :]], axis=-1) / N
            z = jnp.dot(gcat, w1_ref[...], preferred_element_type=jnp.float32)
            z = z + b1_ref[...]
            z = jnp.maximum(z, 0.01 * z)
            lg = jnp.dot(z, w2_ref[...], preferred_element_type=jnp.float32)
            out_ref[...] = lg + b2_ref[...]

    return pl.pallas_call(
        body,
        grid=(nblk,),
        in_specs=[
            pl.BlockSpec((NC, bm, DP), lambda i: (0, i, 0)),
            pl.BlockSpec((2 * D, D), lambda i: (0, 0)),
            pl.BlockSpec((1, D), lambda i: (0, 0)),
            pl.BlockSpec((D, 128), lambda i: (0, 0)),
            pl.BlockSpec((1, 128), lambda i: (0, 0)),
        ],
        out_specs=pl.BlockSpec((1, 128), lambda i: (0, 0)),
        out_shape=jax.ShapeDtypeStruct((1, 128), jnp.float32),
        scratch_shapes=[pltpu.VMEM((2, D), jnp.float32)],
    )(hp, wc1, bc1, wc2p, bc2p)


def kernel(img_x, img_edge_index, img_edge_attr, text_x, text_edge_index,
           text_edge_attr, W0, We0, al0, ar0, ae0, W1, We1, al1, ar1, ae1,
           Wc1, bc1, Wc2, bc2):
    f32 = jnp.float32

    def fold(Wl, all_, arl, Wel, ael):
        Wr = Wl.reshape(Wl.shape[0], H, D)
        Al = jnp.einsum('khd,hd->kh', Wr, all_)
        Ar = jnp.einsum('khd,hd->kh', Wr, arl)
        b2 = jnp.concatenate(
            [Al, Ar, jnp.zeros((Wl.shape[0], 4), f32)], axis=1)
        Ae = jnp.einsum('khd,hd->kh', Wel.reshape(16, H, D), ael)
        return b2.astype(f32), Ae.T.astype(f32)

    b2_0, aet0 = fold(W0, al0, ar0, We0, ae0)
    b2_1, aet1 = fold(W1, al1, ar1, We1, ae1)

    x2 = jnp.concatenate([img_x, text_x], axis=0)
    src2 = jnp.concatenate([img_edge_index[0], text_edge_index[0] + N])
    dst2 = jnp.concatenate([img_edge_index[1], text_edge_index[1]])
    attr2 = jnp.concatenate([img_edge_attr, text_edge_attr], axis=0)

    wc2p = jnp.zeros((D, 128), f32).at[:, :2].set(Wc2)
    bc2p = jnp.zeros((1, 128), f32).at[0, :2].set(bc2)
    bc1r = bc1.reshape(1, D)

    feat, elr = _mm_flat(x2, W0, b2_0)
    featv = feat.reshape(2 * N * 4, DP)
    den, ex = _phase_a(src2, dst2, elr, attr2, aet0)
    hp = _phase_b(src2, dst2, featv, ex, den)

    feat, elr = _mm_planes(hp, W1, b2_1)
    featv = feat.reshape(2 * N * 4, DP)
    den, ex = _phase_a(src2, dst2, elr, attr2, aet1)
    hp = _phase_b(src2, dst2, featv, ex, den)

    out = _final(hp, Wc1, bc1r, wc2p, bc2p)
    return out[:, :2]


# speedup vs baseline: 12.6468x; 12.6468x over previous
import jax
import jax.numpy as jnp
from jax import lax
from jax.experimental import pallas as pl
from jax.experimental.pallas import tpu as pltpu
from jax.experimental.pallas import tpu_sc as plsc

N = 10000
E = 160000
H = 2
D = 256
DP = 128
NC = 2
NS = 16
L = 16
EPT = E // NS
KB = 80
NBLK = EPT // KB
RPT = 640
RPT_LAST = N - 15 * RPT
NH = N // 2
PH = 5136
TRASH = 5120
RPH = 320
PAD4N = 40960

_MESH = plsc.VectorSubcoreMesh(core_axis_name="c", subcore_axis_name="s",
                               num_cores=NC, num_subcores=NS)
_CPARAMS = pltpu.CompilerParams(needs_layout_passes=False)


def _zero_rows(ref, nrows, ncols):
    z = jnp.zeros((L,), jnp.float32)

    def body(i, _):
        for c in range(ncols // L):
            ref[i, pl.ds(c * L, L)] = z
        return 0

    lax.fori_loop(0, nrows, body, 0)


def _phase_a1(src2, dst2, elr, attr2, aet):
    def body(src_hbm, dst_hbm, elr_hbm, attr_hbm, aet_hbm,
             s0_hbm, s1_hbm,
             elf_sh, elrb, rbuf, fbuf, sidx, didx, attrb,
             s0b, s1b, aeb, sem):
        g = lax.axis_index("c")
        sid = lax.axis_index("s")
        iota = lax.iota(jnp.int32, L)
        c0 = jnp.zeros((L,), jnp.int32)
        c1 = c0 + 1
        gN = g * N

        def repack(z):
            pltpu.sync_copy(
                elr_hbm.at[pl.ds(gN + sid * RPT + z * KB, KB), :], rbuf)
            for q in range(KB // L):
                jq = iota + q * L
                for cc in range(4):
                    v = plsc.load_gather(rbuf, [jq, c0 + cc])
                    plsc.store_scatter(fbuf, [jq * 4 + cc], v)
            pltpu.sync_copy(
                fbuf, elf_sh.at[pl.ds((sid * RPT + z * KB) * 4, KB * 4)])

        @pl.when(sid < 15)
        def _():
            for z in range(RPT // KB):
                repack(z)

        @pl.when(sid == 15)
        def _():
            for z in range(RPT_LAST // KB):
                repack(z)

        pltpu.sync_copy(aet_hbm, aeb)
        plsc.subcore_barrier()
        pltpu.sync_copy(elf_sh, elrb)

        ae0s = [plsc.load_gather(aeb, [c0, c0 + dd]) for dd in range(16)]
        ae1s = [plsc.load_gather(aeb, [c1, c0 + dd]) for dd in range(16)]

        def block(b, _):
            base = g * E + sid * EPT + b * KB
            pltpu.sync_copy(src_hbm.at[pl.ds(base, KB)], sidx)
            pltpu.sync_copy(dst_hbm.at[pl.ds(base, KB)], didx)
            pltpu.sync_copy(attr_hbm.at[pl.ds(base, KB), :], attrb)
            for q in range(KB // L):
                jq = iota + q * L
                ee0 = jnp.zeros((L,), jnp.float32)
                ee1 = jnp.zeros((L,), jnp.float32)
                for dd in range(16):
                    col = plsc.load_gather(attrb, [jq, c0 + dd])
                    ee0 = ee0 + col * ae0s[dd]
                    ee1 = ee1 + col * ae1s[dd]
                srcf = (sidx[pl.ds(q * L, L)] - gN) * 4
                dstf = didx[pl.ds(q * L, L)] * 4
                el0 = plsc.load_gather(elrb, [srcf])
                el1 = plsc.load_gather(elrb, [srcf + 1])
                er0 = plsc.load_gather(elrb, [dstf + 2])
                er1 = plsc.load_gather(elrb, [dstf + 3])
                s0b[pl.ds(q * L, L)] = el0 + er0 + ee0
                s1b[pl.ds(q * L, L)] = el1 + er1 + ee1
            pltpu.sync_copy(s0b, s0_hbm.at[pl.ds(base, KB)])
            pltpu.sync_copy(s1b, s1_hbm.at[pl.ds(base, KB)])
            return 0

        lax.fori_loop(0, NBLK, block, 0)

    f = pl.kernel(
        body,
        out_type=[
            jax.ShapeDtypeStruct((2 * E,), jnp.float32),
            jax.ShapeDtypeStruct((2 * E,), jnp.float32),
        ],
        mesh=_MESH,
        compiler_params=_CPARAMS,
        scratch_types=[
            pltpu.VMEM_SHARED((4 * N,), jnp.float32),
            pltpu.VMEM((4 * N,), jnp.float32),
            pltpu.VMEM((KB, 8), jnp.float32),
            pltpu.VMEM((KB * 4,), jnp.float32),
            pltpu.VMEM((KB,), jnp.int32),
            pltpu.VMEM((KB,), jnp.int32),
            pltpu.VMEM((KB, L), jnp.float32),
            pltpu.VMEM((KB,), jnp.float32),
            pltpu.VMEM((KB,), jnp.float32),
            pltpu.VMEM((2, L), jnp.float32),
            pltpu.SemaphoreType.DMA,
        ],
    )
    return f(src2, dst2, elr, attr2, aet)


def _phase_a2(dst2, ex0, ex1):
    def body(dst_hbm, ex0_hbm, ex1_hbm, den_hbm,
             den_sh, didx, didxa, ex0b, ex1b, exw, dcomp, sem):
        dchunk = exw
        g = lax.axis_index("c")
        sid = lax.axis_index("s")
        iota = lax.iota(jnp.int32, L)
        c0 = jnp.zeros((L,), jnp.int32)
        c1 = c0 + 1
        gN = g * N

        for p in range(2):
            lo = p * NH
            _zero_rows(exw, KB, DP)
            for z in range(RPH // KB):
                pltpu.sync_copy(
                    exw, den_sh.at[pl.ds(sid * RPH + z * KB, KB), :])
            plsc.subcore_barrier()

            def block(b, _):
                base = g * E + sid * EPT + b * KB
                pltpu.sync_copy(dst_hbm.at[pl.ds(base, KB)], didx)
                pltpu.sync_copy(ex0_hbm.at[pl.ds(base, KB)], ex0b)
                pltpu.sync_copy(ex1_hbm.at[pl.ds(base, KB)], ex1b)
                for q in range(KB // L):
                    sl = pl.ds(q * L, L)
                    jq = iota + q * L
                    plsc.store_scatter(exw, [jq, c0], ex0b[sl])
                    plsc.store_scatter(exw, [jq, c1], ex1b[sl])
                    dstl = didx[sl] - lo
                    ok = (dstl >= 0) & (dstl < NH)
                    didxa[sl] = jnp.where(ok, dstl, TRASH)
                pltpu.sync_copy(exw, den_sh.at[didxa], add=True)
                return 0

            lax.fori_loop(0, NBLK, block, 0)
            plsc.subcore_barrier()

            for z in range(RPH // KB):
                pltpu.sync_copy(
                    den_sh.at[pl.ds(sid * RPH + z * KB, KB), :], dchunk)
                for q in range(KB // L):
                    jq = iota + q * L
                    d0 = plsc.load_gather(dchunk, [jq, c0])
                    d1 = plsc.load_gather(dchunk, [jq, c1])
                    jf = (jq + z * KB) * 2
                    plsc.store_scatter(dcomp, [jf], d0)
                    plsc.store_scatter(dcomp, [jf + 1], d1)

            @pl.when(sid < 15)
            def _():
                pltpu.sync_copy(
                    dcomp,
                    den_hbm.at[pl.ds((gN + lo + sid * RPH) * 2, RPH * 2)])

            @pl.when(sid == 15)
            def _():
                pltpu.sync_copy(
                    dcomp.at[pl.ds(0, (NH - 15 * RPH) * 2)],
                    den_hbm.at[pl.ds((gN + lo + 15 * RPH) * 2,
                                     (NH - 15 * RPH) * 2)])

    f = pl.kernel(
        body,
        out_type=jax.ShapeDtypeStruct((PAD4N,), jnp.float32),
        mesh=_MESH,
        compiler_params=_CPARAMS,
        scratch_types=[
            pltpu.VMEM_SHARED((PH, DP), jnp.float32),
            pltpu.VMEM((KB,), jnp.int32),
            pltpu.VMEM((KB,), jnp.int32),
            pltpu.VMEM((KB,), jnp.float32),
            pltpu.VMEM((KB,), jnp.float32),
            pltpu.VMEM((KB, DP), jnp.float32),
            pltpu.VMEM((RPH * 2,), jnp.float32),
            pltpu.SemaphoreType.DMA,
        ],
    )
    return f(dst2, ex0, ex1)


def _phase_a3(dst2, ex0, ex1, inv):
    def body(dst_hbm, ex0_hbm, ex1_hbm, inv_hbm,
             a0_hbm, a1_hbm,
             invb, didx, ex0b, ex1b, sem):
        g = lax.axis_index("c")
        sid = lax.axis_index("s")
        gN = g * N
        pltpu.sync_copy(inv_hbm.at[pl.ds(gN * 2, 2 * N)], invb)

        def block(b, _):
            base = g * E + sid * EPT + b * KB
            pltpu.sync_copy(dst_hbm.at[pl.ds(base, KB)], didx)
            pltpu.sync_copy(ex0_hbm.at[pl.ds(base, KB)], ex0b)
            pltpu.sync_copy(ex1_hbm.at[pl.ds(base, KB)], ex1b)
            for q in range(KB // L):
                sl = pl.ds(q * L, L)
                dstf = didx[sl] * 2
                i0 = plsc.load_gather(invb, [dstf])
                i1 = plsc.load_gather(invb, [dstf + 1])
                ex0b[sl] = ex0b[sl] * i0
                ex1b[sl] = ex1b[sl] * i1
            pltpu.sync_copy(ex0b, a0_hbm.at[pl.ds(base, KB)])
            pltpu.sync_copy(ex1b, a1_hbm.at[pl.ds(base, KB)])
            return 0

        lax.fori_loop(0, NBLK, block, 0)

    f = pl.kernel(
        body,
        out_type=[
            jax.ShapeDtypeStruct((2 * E,), jnp.float32),
            jax.ShapeDtypeStruct((2 * E,), jnp.float32),
        ],
        mesh=_MESH,
        compiler_params=_CPARAMS,
        scratch_types=[
            pltpu.VMEM((2 * N,), jnp.float32),
            pltpu.VMEM((KB,), jnp.int32),
            pltpu.VMEM((KB,), jnp.float32),
            pltpu.VMEM((KB,), jnp.float32),
            pltpu.SemaphoreType.DMA,
        ],
    )
    return f(dst2, ex0, ex1, inv)


def _tc_exp(s0, s1):
    def body(s0_ref, s1_ref, e0_ref, e1_ref):
        v0 = s0_ref[...]
        e0_ref[...] = jnp.exp(jnp.maximum(v0, 0.2 * v0))
        v1 = s1_ref[...]
        e1_ref[...] = jnp.exp(jnp.maximum(v1, 0.2 * v1))

    r0 = s0.reshape(2 * E // 128, 128)
    r1 = s1.reshape(2 * E // 128, 128)
    bm = 2 * E // 128
    e0, e1 = pl.pallas_call(
        body,
        grid=(1,),
        in_specs=[pl.BlockSpec((bm, 128), lambda i: (0, 0)),
                  pl.BlockSpec((bm, 128), lambda i: (0, 0))],
        out_specs=[pl.BlockSpec((bm, 128), lambda i: (0, 0)),
                   pl.BlockSpec((bm, 128), lambda i: (0, 0))],
        out_shape=[jax.ShapeDtypeStruct(r0.shape, jnp.float32),
                   jax.ShapeDtypeStruct(r1.shape, jnp.float32)],
    )(r0, r1)
    return e0.reshape(2 * E), e1.reshape(2 * E)


def _tc_inv(den):
    def body(d_ref, i_ref):
        i_ref[...] = 0.5 / (d_ref[...] + 1e-9)

    r = den.reshape(PAD4N // 128, 128)
    out = pl.pallas_call(
        body,
        grid=(1,),
        in_specs=[pl.BlockSpec(r.shape, lambda i: (0, 0))],
        out_specs=pl.BlockSpec(r.shape, lambda i: (0, 0)),
        out_shape=jax.ShapeDtypeStruct(r.shape, jnp.float32),
    )(r)
    return out.reshape(PAD4N)


def _phase_b(src2, dst2, featv, a0, a1):
    def body(src_hbm, dst_hbm, feat_hbm, a0_hbm, a1_hbm,
             h_hbm,
             acc_sh, sidx, didx, fi0, fi1,
             f0b, f1b, a0b, a1b, msg, sem):
        p = lax.axis_index("c")
        sid = lax.axis_index("s")
        c0 = jnp.zeros((L,), jnp.int32)

        for g in range(2):
            _zero_rows(msg, KB, DP)

            @pl.when(sid < 15)
            def _():
                for z in range(RPT // KB):
                    pltpu.sync_copy(
                        msg, acc_sh.at[pl.ds(sid * RPT + z * KB, KB), :])

            @pl.when(sid == 15)
            def _():
                for z in range(RPT_LAST // KB):
                    pltpu.sync_copy(
                        msg, acc_sh.at[pl.ds(15 * RPT + z * KB, KB), :])

            plsc.subcore_barrier()

            def block(b, _):
                base = g * E + sid * EPT + b * KB
                pltpu.sync_copy(src_hbm.at[pl.ds(base, KB)], sidx)
                pltpu.sync_copy(dst_hbm.at[pl.ds(base, KB)], didx)
                pltpu.sync_copy(a0_hbm.at[pl.ds(base, KB)], a0b)
                pltpu.sync_copy(a1_hbm.at[pl.ds(base, KB)], a1b)
                for q in range(KB // L):
                    sv = sidx[pl.ds(q * L, L)]
                    f0 = sv * 4 + p
                    fi0[pl.ds(q * L, L)] = f0
                    fi1[pl.ds(q * L, L)] = f0 + 2
                pltpu.async_copy(feat_hbm.at[fi0], f0b, sem).wait()
                pltpu.async_copy(feat_hbm.at[fi1], f1b, sem).wait()

                def edge(j, _):
                    js = c0 + j
                    av0 = plsc.load_gather(a0b, [js])
                    av1 = plsc.load_gather(a1b, [js])
                    for c in range(DP // L):
                        sl = pl.ds(c * L, L)
                        msg[j, sl] = av0 * f0b[j, sl] + av1 * f1b[j, sl]
                    return 0

                lax.fori_loop(0, KB, edge, 0)
                pltpu.sync_copy(msg, acc_sh.at[didx], add=True)
                return 0

            lax.fori_loop(0, NBLK, block, 0)
            plsc.subcore_barrier()

            @pl.when(sid < 15)
            def _():
                pltpu.sync_copy(
                    acc_sh.at[pl.ds(sid * RPT, RPT), :],
                    h_hbm.at[p, pl.ds(g * N + sid * RPT, RPT), :])

            @pl.when(sid == 15)
            def _():
                pltpu.sync_copy(
                    acc_sh.at[pl.ds(15 * RPT, RPT_LAST), :],
                    h_hbm.at[p, pl.ds(g * N + 15 * RPT, RPT_LAST), :])

            plsc.subcore_barrier()

    f = pl.kernel(
        body,
        out_type=jax.ShapeDtypeStruct((NC, 2 * N, DP), jnp.float32),
        mesh=_MESH,
        compiler_params=_CPARAMS,
        scratch_types=[
            pltpu.VMEM_SHARED((N, DP), jnp.float32),
            pltpu.VMEM((KB,), jnp.int32),
            pltpu.VMEM((KB,), jnp.int32),
            pltpu.VMEM((KB,), jnp.int32),
            pltpu.VMEM((KB,), jnp.int32),
            pltpu.VMEM((KB, DP), jnp.float32),
            pltpu.VMEM((KB, DP), jnp.float32),
            pltpu.VMEM((KB,), jnp.float32),
            pltpu.VMEM((KB,), jnp.float32),
            pltpu.VMEM((KB, DP), jnp.float32),
            pltpu.SemaphoreType.DMA,
        ],
    )
    return f(src2, dst2, featv, a0, a1)


def _mm_flat(x2, wb, b2, bm=2000):
    n = x2.shape[0]

    def body(x_ref, w_ref, b2_ref, feat_ref, elr_ref):
        x = x_ref[...]
        feat_ref[...] = jnp.dot(x, w_ref[...], preferred_element_type=jnp.float32)
        elr_ref[...] = jnp.dot(x, b2_ref[...], preferred_element_type=jnp.float32)

    return pl.pallas_call(
        body,
        grid=(n // bm,),
        in_specs=[
            pl.BlockSpec((bm, D), lambda i: (i, 0)),
            pl.BlockSpec((D, H * D), lambda i: (0, 0)),
            pl.BlockSpec((D, 8), lambda i: (0, 0)),
        ],
        out_specs=[
            pl.BlockSpec((bm, H * D), lambda i: (i, 0)),
            pl.BlockSpec((bm, 8), lambda i: (i, 0)),
        ],
        out_shape=[
            jax.ShapeDtypeStruct((n, H * D), jnp.float32),
            jax.ShapeDtypeStruct((n, 8), jnp.float32),
        ],
    )(x2, wb, b2)


def _mm_planes(hp, wb, b2, bm=2000):
    n = hp.shape[1]

    def body(x_ref, w_ref, b2_ref, feat_ref, elr_ref):
        x = jnp.concatenate([x_ref[0], x_ref[1]], axis=-1)
        feat_ref[...] = jnp.dot(x, w_ref[...], preferred_element_type=jnp.float32)
        elr_ref[...] = jnp.dot(x, b2_ref[...], preferred_element_type=jnp.float32)

    return pl.pallas_call(
        body,
        grid=(n // bm,),
        in_specs=[
            pl.BlockSpec((NC, bm, DP), lambda i: (0, i, 0)),
            pl.BlockSpec((D, H * D), lambda i: (0, 0)),
            pl.BlockSpec((D, 8), lambda i: (0, 0)),
        ],
        out_specs=[
            pl.BlockSpec((bm, H * D), lambda i: (i, 0)),
            pl.BlockSpec((bm, 8), lambda i: (i, 0)),
        ],
        out_shape=[
            jax.ShapeDtypeStruct((n, H * D), jnp.float32),
            jax.ShapeDtypeStruct((n, 8), jnp.float32),
        ],
    )(hp, wb, b2)


def _final(hp, wc1, bc1, wc2p, bc2p, bm=2000):
    nblk = 2 * N // bm
    half = nblk // 2

    def body(x_ref, w1_ref, b1_ref, w2_ref, b2_ref, out_ref, acc):
        i = pl.program_id(0)

        @pl.when(i == 0)
        def _():
            acc[...] = jnp.zeros_like(acc)

        x = jnp.concatenate([x_ref[0], x_ref[1]], axis=-1)
        s = jnp.sum(x, axis=0, keepdims=True)
        is_img = (i < half).astype(jnp.float32)
        acc[0:1, :] += is_img * s
        acc[1:2, :] += (1.0 - is_img) * s

        @pl.when(i == nblk - 1)
        def _():
            gcat = jnp.concatenate([acc[0:1, :], acc[1:2, :]], axis=-1) / N
            z = jnp.dot(gcat, w1_ref[...], preferred_element_type=jnp.float32)
            z = z + b1_ref[...]
            z = jnp.maximum(z, 0.01 * z)
            lg = jnp.dot(z, w2_ref[...], preferred_element_type=jnp.float32)
            out_ref[...] = lg + b2_ref[...]

    return pl.pallas_call(
        body,
        grid=(nblk,),
        in_specs=[
            pl.BlockSpec((NC, bm, DP), lambda i: (0, i, 0)),
            pl.BlockSpec((2 * D, D), lambda i: (0, 0)),
            pl.BlockSpec((1, D), lambda i: (0, 0)),
            pl.BlockSpec((D, 128), lambda i: (0, 0)),
            pl.BlockSpec((1, 128), lambda i: (0, 0)),
        ],
        out_specs=pl.BlockSpec((1, 128), lambda i: (0, 0)),
        out_shape=jax.ShapeDtypeStruct((1, 128), jnp.float32),
        scratch_shapes=[pltpu.VMEM((2, D), jnp.float32)],
    )(hp, wc1, bc1, wc2p, bc2p)


def kernel(img_x, img_edge_index, img_edge_attr, text_x, text_edge_index,
           text_edge_attr, W0, We0, al0, ar0, ae0, W1, We1, al1, ar1, ae1,
           Wc1, bc1, Wc2, bc2):
    f32 = jnp.float32

    def fold(Wl, all_, arl, Wel, ael):
        Wr = Wl.reshape(Wl.shape[0], H, D)
        Al = jnp.einsum('khd,hd->kh', Wr, all_)
        Ar = jnp.einsum('khd,hd->kh', Wr, arl)
        b2 = jnp.concatenate(
            [Al, Ar, jnp.zeros((Wl.shape[0], 4), f32)], axis=1)
        Ae = jnp.einsum('khd,hd->kh', Wel.reshape(16, H, D), ael)
        return b2.astype(f32), Ae.T.astype(f32)

    b2_0, aet0 = fold(W0, al0, ar0, We0, ae0)
    b2_1, aet1 = fold(W1, al1, ar1, We1, ae1)

    x2 = jnp.concatenate([img_x, text_x], axis=0)
    src2 = jnp.concatenate([img_edge_index[0], text_edge_index[0] + N])
    dst2 = jnp.concatenate([img_edge_index[1], text_edge_index[1]])
    attr2 = jnp.concatenate([img_edge_attr, text_edge_attr], axis=0)

    wc2p = jnp.zeros((D, 128), f32).at[:, :2].set(Wc2)
    bc2p = jnp.zeros((1, 128), f32).at[0, :2].set(bc2)
    bc1r = bc1.reshape(1, D)

    feat, elr = _mm_flat(x2, W0, b2_0)
    featv = feat.reshape(2 * N * 4, DP)
    s0, s1 = _phase_a1(src2, dst2, elr, attr2, aet0)
    ex0, ex1 = _tc_exp(s0, s1)
    den = _phase_a2(dst2, ex0, ex1)
    inv = _tc_inv(den)
    a0, a1 = _phase_a3(dst2, ex0, ex1, inv)
    hp = _phase_b(src2, dst2, featv, a0, a1)

    feat, elr = _mm_planes(hp, W1, b2_1)
    featv = feat.reshape(2 * N * 4, DP)
    s0, s1 = _phase_a1(src2, dst2, elr, attr2, aet1)
    ex0, ex1 = _tc_exp(s0, s1)
    den = _phase_a2(dst2, ex0, ex1)
    inv = _tc_inv(den)
    a0, a1 = _phase_a3(dst2, ex0, ex1, inv)
    hp = _phase_b(src2, dst2, featv, a0, a1)

    out = _final(hp, Wc1, bc1r, wc2p, bc2p)
    return out[:, :2]
